# baseline (speedup 1.0000x reference)
import dataclasses
import functools

import jax
import jax.numpy as jnp
from jax import lax
from jax.experimental import pallas as pl
from jax.experimental.pallas import tpu as pltpu
from jax.experimental.pallas import tpu_sc as plsc


@functools.lru_cache(maxsize=None)
def _fle_kernel(B, L, D):
    info = plsc.get_sparse_core_info()
    NC, NS, LN = info.num_cores, info.num_subcores, info.num_lanes
    NW = NC * NS
    C = 32
    b_per_w = B // NW
    n_chunks = b_per_w // C
    nfull = L // LN
    rem = L - nfull * LN
    W = 2 * D

    mesh = plsc.VectorSubcoreMesh(core_axis_name="c", subcore_axis_name="s")
    cp = pltpu.CompilerParams(use_tc_tiling_on_sc=True)
    if "needs_layout_passes" in pltpu.CompilerParams.__dataclass_fields__:
        cp = dataclasses.replace(cp, needs_layout_passes=False)

    @functools.partial(
        pl.kernel,
        out_type=jax.ShapeDtypeStruct((B, W), jnp.float32),
        mesh=mesh,
        compiler_params=cp,
        scratch_types=[
            pltpu.VMEM((C, L), jnp.int32),
            pltpu.VMEM((C, 8, W), jnp.float32),
            pltpu.VMEM((C, 8, W), jnp.float32),
            pltpu.VMEM((C, W), jnp.float32),
            pltpu.SemaphoreType.DMA,
            pltpu.SemaphoreType.DMA,
            pltpu.SemaphoreType.DMA,
        ],
    )
    def k(mask_hbm, tokw_hbm, out_hbm, mask_v, firstg_v, lastg_v, comb_v,
          semm, semf, seml):
        wid = lax.axis_index("s") * NC + lax.axis_index("c")
        lanes = lax.iota(jnp.int32, LN)
        tail_keep = (lanes >= LN - rem).astype(jnp.int32)

        @pl.loop(0, n_chunks)
        def _(ch):
            row0 = wid * b_per_w + ch * C
            cpm = pltpu.async_copy(mask_hbm.at[pl.ds(row0, C), :], mask_v,
                                   semm)
            fcopies = []
            s0_list = []
            for r in range(C):
                fi0 = (row0 + r) * L
                blk0 = pl.multiple_of((fi0 >> 4) * 8, 8)
                fcopies.append(
                    pltpu.async_copy(tokw_hbm.at[pl.ds(blk0, 8), :],
                                     firstg_v.at[r], semf))
                s0_list.append((fi0 >> 1) & 7)
            cpm.wait()
            copies = []
            sel_list = []
            for r in range(C):
                acc = mask_v[r, pl.ds(nfull * LN - LN + rem, LN)] * tail_keep
                for cblk in range(nfull):
                    acc = acc + mask_v[r, pl.ds(cblk * LN, LN)]
                ln = jnp.sum(acc)
                adj = jnp.maximum(ln - 1, 0)
                fi = (row0 + r) * L + adj
                blk = pl.multiple_of((fi >> 4) * 8, 8)
                copies.append(
                    pltpu.async_copy(tokw_hbm.at[pl.ds(blk, 8), :],
                                     lastg_v.at[r], seml))
                sel_list.append(((fi >> 1) & 7, (fi & 1) * D))
            for cp_ in fcopies:
                cp_.wait()
            for r in range(C):
                s0 = s0_list[r]
                for c in range(0, D, LN):
                    comb_v[r, pl.ds(c, LN)] = firstg_v[r, s0, pl.ds(c, LN)]
            for cp_ in copies:
                cp_.wait()
            for r in range(C):
                js, hoff = sel_list[r]
                for c in range(0, D, LN):
                    comb_v[r, pl.ds(D + c, LN)] = (
                        lastg_v[r, js, pl.ds(hoff + c, LN)])
            pltpu.sync_copy(comb_v, out_hbm.at[pl.ds(row0, C), :])

    return k


def kernel(mask, embedded_tokens):
    B, L, D = embedded_tokens.shape
    tokw = embedded_tokens.reshape(B * L // 2, 2 * D)
    return _fle_kernel(B, L, D)(mask.astype(jnp.int32), tokw)


# speedup vs baseline: 2.8250x; 2.8250x over previous
import dataclasses
import functools

import jax
import jax.numpy as jnp
from jax import lax
from jax.experimental import pallas as pl
from jax.experimental.pallas import tpu as pltpu
from jax.experimental.pallas import tpu_sc as plsc


@functools.lru_cache(maxsize=None)
def _lens_sc_kernel(L, B):
    info = plsc.get_sparse_core_info()
    NC, NS, LN = info.num_cores, info.num_subcores, info.num_lanes
    NW = NC * NS
    b_per_w = B // NW
    groups = b_per_w // LN

    mesh = plsc.VectorSubcoreMesh(core_axis_name="c", subcore_axis_name="s")
    cp = pltpu.CompilerParams(use_tc_tiling_on_sc=True)
    if "needs_layout_passes" in pltpu.CompilerParams.__dataclass_fields__:
        cp = dataclasses.replace(cp, needs_layout_passes=False)

    @functools.partial(
        pl.kernel,
        out_type=jax.ShapeDtypeStruct((B,), jnp.int32),
        mesh=mesh,
        compiler_params=cp,
        scratch_types=[
            pltpu.VMEM((L, b_per_w), jnp.int32),
            pltpu.VMEM((b_per_w,), jnp.int32),
            pltpu.SemaphoreType.DMA,
        ],
    )
    def k(maskt_hbm, adj_hbm, mask_v, adj_v, semm):
        wid = lax.axis_index("s") * NC + lax.axis_index("c")
        b0 = wid * b_per_w
        pltpu.async_copy(maskt_hbm.at[:, pl.ds(b0, b_per_w)], mask_v,
                         semm).wait()
        for g in range(groups):
            def step(l, acc, g=g):
                return acc + mask_v[l, pl.ds(g * LN, LN)]

            lens = lax.fori_loop(0, L, step, jnp.zeros((LN,), jnp.int32))
            adj_v[pl.ds(g * LN, LN)] = jnp.maximum(lens - 1, 0)
        pltpu.sync_copy(adj_v, adj_hbm.at[pl.ds(b0, b_per_w)])

    return k


@functools.lru_cache(maxsize=None)
def _select_tc_kernel(L, D, B):
    LB = 8
    BB = 512
    nl = L // LB
    nb = B // BB

    def body(adj_ref, tok_ref, out_ref):
        li = pl.program_id(1)
        adjb = adj_ref[0, :]

        @pl.when(li == 0)
        def _():
            out_ref[0:D, :] = tok_ref[0]
            out_ref[D:2 * D, :] = jnp.zeros((D, BB), jnp.float32)

        acc = out_ref[D:2 * D, :]
        lt = li * LB
        for s in range(LB):
            sel = (adjb == lt + s)[None, :]
            acc = jnp.where(sel, tok_ref[s], acc)
        out_ref[D:2 * D, :] = acc

    return pl.pallas_call(
        body,
        grid=(nb, nl),
        in_specs=[
            pl.BlockSpec((1, BB), lambda b, l: (0, b)),
            pl.BlockSpec((LB, D, BB), lambda b, l: (l, 0, b)),
        ],
        out_specs=pl.BlockSpec((2 * D, BB), lambda b, l: (0, b)),
        out_shape=jax.ShapeDtypeStruct((2 * D, B), jnp.float32),
        compiler_params=pltpu.CompilerParams(
            dimension_semantics=("parallel", "arbitrary")),
    )


def kernel(mask, embedded_tokens):
    B, L, D = embedded_tokens.shape
    maskt = mask.astype(jnp.int32).T
    tokt = jnp.transpose(embedded_tokens, (1, 2, 0))
    adj = _lens_sc_kernel(L, B)(maskt)
    outt = _select_tc_kernel(L, D, B)(adj.reshape(1, B), tokt)
    return outt.T


# speedup vs baseline: 5.2959x; 1.8747x over previous
import dataclasses
import functools

import jax
import jax.numpy as jnp
from jax import lax
from jax.experimental import pallas as pl
from jax.experimental.pallas import tpu as pltpu
from jax.experimental.pallas import tpu_sc as plsc


@functools.lru_cache(maxsize=None)
def _lens_sc_kernel(L, B):
    info = plsc.get_sparse_core_info()
    NC, NS, LN = info.num_cores, info.num_subcores, info.num_lanes
    NW = NC * NS
    b_per_w = B // NW
    groups = b_per_w // LN

    mesh = plsc.VectorSubcoreMesh(core_axis_name="c", subcore_axis_name="s")
    cp = pltpu.CompilerParams(use_tc_tiling_on_sc=True)
    if "needs_layout_passes" in pltpu.CompilerParams.__dataclass_fields__:
        cp = dataclasses.replace(cp, needs_layout_passes=False)

    @functools.partial(
        pl.kernel,
        out_type=jax.ShapeDtypeStruct((B,), jnp.int32),
        mesh=mesh,
        compiler_params=cp,
        scratch_types=[
            pltpu.VMEM((L, b_per_w), jnp.int32),
            pltpu.VMEM((b_per_w,), jnp.int32),
            pltpu.SemaphoreType.DMA,
        ],
    )
    def k(maskt_hbm, adj_hbm, mask_v, adj_v, semm):
        wid = lax.axis_index("s") * NC + lax.axis_index("c")
        b0 = wid * b_per_w
        pltpu.async_copy(maskt_hbm.at[:, pl.ds(b0, b_per_w)], mask_v,
                         semm).wait()
        for g in range(groups):
            def step(l, acc, g=g):
                return acc + mask_v[l, pl.ds(g * LN, LN)]

            lens = lax.fori_loop(0, L, step, jnp.zeros((LN,), jnp.int32))
            adj_v[pl.ds(g * LN, LN)] = jnp.maximum(lens - 1, 0)
        pltpu.sync_copy(adj_v, adj_hbm.at[pl.ds(b0, b_per_w)])

    return k


@functools.lru_cache(maxsize=None)
def _select_tc_kernel(L, D, B):
    LB = 8
    BB = 2048
    nl = L // LB
    nb = B // BB

    def body(adj_ref, tok_ref, out_ref, acc_ref):
        li = pl.program_id(1)
        adjb = adj_ref[0, :]

        @pl.when(li == 0)
        def _():
            acc_ref[0:D, :] = tok_ref[0]
            acc_ref[D:2 * D, :] = jnp.zeros((D, BB), jnp.float32)

        acc = acc_ref[D:2 * D, :]
        lt = li * LB
        for s in range(LB):
            sel = (adjb == lt + s)[None, :]
            acc = jnp.where(sel, tok_ref[s], acc)
        acc_ref[D:2 * D, :] = acc

        @pl.when(li == nl - 1)
        def _():
            out_ref[...] = acc_ref[...].T

    return pl.pallas_call(
        body,
        grid=(nb, nl),
        in_specs=[
            pl.BlockSpec((1, BB), lambda b, l: (0, b)),
            pl.BlockSpec((LB, D, BB), lambda b, l: (l, 0, b)),
        ],
        out_specs=pl.BlockSpec((BB, 2 * D), lambda b, l: (b, 0)),
        out_shape=jax.ShapeDtypeStruct((B, 2 * D), jnp.float32),
        scratch_shapes=[pltpu.VMEM((2 * D, BB), jnp.float32)],
        compiler_params=pltpu.CompilerParams(
            dimension_semantics=("parallel", "arbitrary")),
    )


def kernel(mask, embedded_tokens):
    B, L, D = embedded_tokens.shape
    maskt = mask.astype(jnp.int32).T
    tokt = jnp.transpose(embedded_tokens, (1, 2, 0))
    adj = _lens_sc_kernel(L, B)(maskt)
    return _select_tc_kernel(L, D, B)(adj.reshape(1, B), tokt)


# speedup vs baseline: 6.9520x; 1.3127x over previous
import dataclasses
import functools

import jax
import jax.numpy as jnp
from jax import lax
from jax.experimental import pallas as pl
from jax.experimental.pallas import tpu as pltpu
from jax.experimental.pallas import tpu_sc as plsc


@functools.lru_cache(maxsize=None)
def _lens_sc_kernel(L, B):
    info = plsc.get_sparse_core_info()
    NC, NS, LN = info.num_cores, info.num_subcores, info.num_lanes
    NW = NC * NS
    b_per_w = B // NW
    groups = b_per_w // LN

    mesh = plsc.VectorSubcoreMesh(core_axis_name="c", subcore_axis_name="s")
    cp = pltpu.CompilerParams(use_tc_tiling_on_sc=True)
    if "needs_layout_passes" in pltpu.CompilerParams.__dataclass_fields__:
        cp = dataclasses.replace(cp, needs_layout_passes=False)

    @functools.partial(
        pl.kernel,
        out_type=(jax.ShapeDtypeStruct((B,), jnp.int32),
                  jax.ShapeDtypeStruct((NW * LN,), jnp.int32),
                  jax.ShapeDtypeStruct((NW * LN,), jnp.int32)),
        mesh=mesh,
        compiler_params=cp,
        scratch_types=[
            pltpu.VMEM((L, b_per_w), jnp.int32),
            pltpu.VMEM((b_per_w,), jnp.int32),
            pltpu.VMEM((LN,), jnp.int32),
            pltpu.VMEM((LN,), jnp.int32),
            pltpu.SemaphoreType.DMA,
        ],
    )
    def k(maskt_hbm, adj_hbm, min_hbm, max_hbm, mask_v, adj_v, min_v, max_v,
          semm):
        wid = lax.axis_index("s") * NC + lax.axis_index("c")
        b0 = wid * b_per_w
        pltpu.async_copy(maskt_hbm.at[:, pl.ds(b0, b_per_w)], mask_v,
                         semm).wait()
        vmin = jnp.zeros((LN,), jnp.int32) + (L - 1)
        vmax = jnp.zeros((LN,), jnp.int32)
        for g in range(groups):
            def step(l, acc, g=g):
                return acc + mask_v[l, pl.ds(g * LN, LN)]

            lens = lax.fori_loop(0, L, step, jnp.zeros((LN,), jnp.int32))
            adj = jnp.maximum(lens - 1, 0)
            adj_v[pl.ds(g * LN, LN)] = adj
            vmin = jnp.minimum(vmin, adj)
            vmax = jnp.maximum(vmax, adj)
        min_v[...] = vmin
        max_v[...] = vmax
        pltpu.sync_copy(adj_v, adj_hbm.at[pl.ds(b0, b_per_w)])
        pltpu.sync_copy(min_v, min_hbm.at[pl.ds(wid * LN, LN)])
        pltpu.sync_copy(max_v, max_hbm.at[pl.ds(wid * LN, LN)])

    return k


@functools.lru_cache(maxsize=None)
def _select_tc_kernel(L, D, B):
    LB = 8
    BB = 2048
    nl = L // LB
    nb = B // BB

    def body(lmap_ref, adj_ref, tok_ref, out_ref, acc_ref):
        li = pl.program_id(1)
        adjb = adj_ref[0, :]

        @pl.when(li == 0)
        def _():
            acc_ref[0:D, :] = tok_ref[0]
            acc_ref[D:2 * D, :] = jnp.zeros((D, BB), jnp.float32)

        acc = acc_ref[D:2 * D, :]
        lt = lmap_ref[pl.program_id(0) * nl + li] * LB
        for s in range(LB):
            sel = (adjb == lt + s)[None, :]
            acc = jnp.where(sel, tok_ref[s], acc)
        acc_ref[D:2 * D, :] = acc

        @pl.when(li == nl - 1)
        def _():
            out_ref[...] = acc_ref[...].T

    grid_spec = pltpu.PrefetchScalarGridSpec(
        num_scalar_prefetch=1,
        grid=(nb, nl),
        in_specs=[
            pl.BlockSpec((1, BB), lambda b, l, lm: (0, b)),
            pl.BlockSpec((LB, D, BB), lambda b, l, lm: (lm[b * nl + l], 0, b)),
        ],
        out_specs=pl.BlockSpec((BB, 2 * D), lambda b, l, lm: (b, 0)),
        scratch_shapes=[pltpu.VMEM((2 * D, BB), jnp.float32)],
    )
    return pl.pallas_call(
        body,
        grid_spec=grid_spec,
        out_shape=jax.ShapeDtypeStruct((B, 2 * D), jnp.float32),
        compiler_params=pltpu.CompilerParams(
            dimension_semantics=("parallel", "arbitrary")),
    )


def kernel(mask, embedded_tokens):
    B, L, D = embedded_tokens.shape
    LB, BB = 8, 2048
    nl, nb = L // LB, B // BB
    maskt = mask.astype(jnp.int32).T
    tokt = jnp.transpose(embedded_tokens, (1, 2, 0))
    adj, vmin, vmax = _lens_sc_kernel(L, B)(maskt)
    wmin = vmin.reshape(-1, 16).min(axis=1)
    wmax = vmax.reshape(-1, 16).max(axis=1)
    w_per_b = wmin.shape[0] // nb
    lminb = wmin.reshape(nb, w_per_b).min(axis=1) // LB
    lmaxb = wmax.reshape(nb, w_per_b).max(axis=1) // LB
    j = jnp.arange(nl, dtype=jnp.int32)
    lstart = jnp.maximum(lminb, 1)
    lmap = jnp.where(j[None, :] == 0, 0,
                     jnp.minimum(lstart[:, None] + (j[None, :] - 1),
                                 lmaxb[:, None])).astype(jnp.int32)
    return _select_tc_kernel(L, D, B)(lmap.reshape(-1), adj.reshape(1, B),
                                      tokt)


# speedup vs baseline: 10.9691x; 1.5778x over previous
import dataclasses
import functools

import jax
import jax.numpy as jnp
from jax import lax
from jax.experimental import pallas as pl
from jax.experimental.pallas import tpu as pltpu
from jax.experimental.pallas import tpu_sc as plsc


@functools.lru_cache(maxsize=None)
def _lens_sc_kernel(L, B):
    info = plsc.get_sparse_core_info()
    NC, NS, LN = info.num_cores, info.num_subcores, info.num_lanes
    NW = NC * NS
    b_per_w = B // NW
    groups = b_per_w // LN

    mesh = plsc.VectorSubcoreMesh(core_axis_name="c", subcore_axis_name="s")
    cp = pltpu.CompilerParams(use_tc_tiling_on_sc=True)
    if "needs_layout_passes" in pltpu.CompilerParams.__dataclass_fields__:
        cp = dataclasses.replace(cp, needs_layout_passes=False)

    @functools.partial(
        pl.kernel,
        out_type=(jax.ShapeDtypeStruct((B,), jnp.int32),
                  jax.ShapeDtypeStruct((NW * LN,), jnp.int32),
                  jax.ShapeDtypeStruct((NW * LN,), jnp.int32)),
        mesh=mesh,
        compiler_params=cp,
        scratch_types=[
            pltpu.VMEM((L, b_per_w), jnp.int32),
            pltpu.VMEM((b_per_w,), jnp.int32),
            pltpu.VMEM((LN,), jnp.int32),
            pltpu.VMEM((LN,), jnp.int32),
            pltpu.SemaphoreType.DMA,
        ],
    )
    def k(maskt_hbm, adj_hbm, min_hbm, max_hbm, mask_v, adj_v, min_v, max_v,
          semm):
        wid = lax.axis_index("s") * NC + lax.axis_index("c")
        b0 = wid * b_per_w
        pltpu.async_copy(maskt_hbm.at[:, pl.ds(b0, b_per_w)], mask_v,
                         semm).wait()
        vmin = jnp.zeros((LN,), jnp.int32) + (L - 1)
        vmax = jnp.zeros((LN,), jnp.int32)
        for g in range(groups):
            def step(l, acc, g=g):
                return acc + mask_v[l, pl.ds(g * LN, LN)]

            lens = lax.fori_loop(0, L, step, jnp.zeros((LN,), jnp.int32))
            adj = jnp.maximum(lens - 1, 0)
            adj_v[pl.ds(g * LN, LN)] = adj
            vmin = jnp.minimum(vmin, adj)
            vmax = jnp.maximum(vmax, adj)
        min_v[...] = vmin
        max_v[...] = vmax
        pltpu.sync_copy(adj_v, adj_hbm.at[pl.ds(b0, b_per_w)])
        pltpu.sync_copy(min_v, min_hbm.at[pl.ds(wid * LN, LN)])
        pltpu.sync_copy(max_v, max_hbm.at[pl.ds(wid * LN, LN)])

    return k


@functools.lru_cache(maxsize=None)
def _select_tc_kernel(L, D, B):
    LB = 8
    BB = 2048
    nl = L // LB
    nb = B // BB

    def body(lmap_ref, adj_ref, tok_ref, out_ref, acc_ref):
        li = pl.program_id(1)
        bi = pl.program_id(0)

        @pl.when(li == 0)
        def _():
            acc_ref[0:D, :] = tok_ref[0]
            acc_ref[D:2 * D, :] = jnp.zeros((D, BB), jnp.float32)

        cur = lmap_ref[bi * nl + li]
        prev = lmap_ref[bi * nl + jnp.maximum(li - 1, 0)]
        @pl.when((li == 0) | (cur != prev))
        def _():
            adjb = adj_ref[0, :]
            acc = acc_ref[D:2 * D, :]
            lt = cur * LB
            for s in range(LB):
                sel = (adjb == lt + s)[None, :]
                acc = jnp.where(sel, tok_ref[s], acc)
            acc_ref[D:2 * D, :] = acc

        @pl.when(li == nl - 1)
        def _():
            out_ref[...] = acc_ref[...].T

    grid_spec = pltpu.PrefetchScalarGridSpec(
        num_scalar_prefetch=1,
        grid=(nb, nl),
        in_specs=[
            pl.BlockSpec((1, BB), lambda b, l, lm: (0, b)),
            pl.BlockSpec((LB, D, BB), lambda b, l, lm: (lm[b * nl + l], 0, b)),
        ],
        out_specs=pl.BlockSpec((BB, 2 * D), lambda b, l, lm: (b, 0)),
        scratch_shapes=[pltpu.VMEM((2 * D, BB), jnp.float32)],
    )
    return pl.pallas_call(
        body,
        grid_spec=grid_spec,
        out_shape=jax.ShapeDtypeStruct((B, 2 * D), jnp.float32),
        compiler_params=pltpu.CompilerParams(
            dimension_semantics=("parallel", "arbitrary")),
    )


def kernel(mask, embedded_tokens):
    B, L, D = embedded_tokens.shape
    LB, BB = 8, 2048
    nl, nb = L // LB, B // BB
    maskt = mask.astype(jnp.int32).T
    tokt = jnp.transpose(embedded_tokens, (1, 2, 0))
    adj, vmin, vmax = _lens_sc_kernel(L, B)(maskt)
    wmin = vmin.reshape(-1, 16).min(axis=1)
    wmax = vmax.reshape(-1, 16).max(axis=1)
    w_per_b = wmin.shape[0] // nb
    lminb = wmin.reshape(nb, w_per_b).min(axis=1) // LB
    lmaxb = wmax.reshape(nb, w_per_b).max(axis=1) // LB
    j = jnp.arange(nl, dtype=jnp.int32)
    lstart = jnp.maximum(lminb, 1)
    lmap = jnp.where(j[None, :] == 0, 0,
                     jnp.minimum(lstart[:, None] + (j[None, :] - 1),
                                 lmaxb[:, None])).astype(jnp.int32)
    return _select_tc_kernel(L, D, B)(lmap.reshape(-1), adj.reshape(1, B),
                                      tokt)
